# Initial kernel scaffold; baseline (speedup 1.0000x reference)
import jax
import jax.numpy as jnp
from jax.experimental import pallas as pl


def kernel(mrope_position_ids_padding, mrope_position_deltas, inv_freq):
    raise NotImplementedError("write your pallas kernel here")


# speedup vs baseline: 25.4352x; 25.4352x over previous
import functools

import jax
import jax.numpy as jnp
from jax import lax
from jax.experimental import pallas as pl
from jax.experimental.pallas import tpu as pltpu
from jax.experimental.pallas import tpu_sc as plsc

MAX_POS = 8192
HEAD_DIM = 128
BATCH = 4
COLS = (32, 48, 48)
COL_OFF = (0, 32, 80)

NC, NS = 2, 16
NW = NC * NS
ROWS = BATCH * MAX_POS
RPW = ROWS // NW
GCH = 128
NG = RPW // GCH
WPB = MAX_POS // RPW


def _table_body(f2a_ref, f2b_ref, f2c_ref, ta_ref, tb_ref, tc_ref):
    i = pl.program_id(0)
    blk = ta_ref.shape[0]
    rows = lax.broadcasted_iota(jnp.int32, (blk, 1), 0) + i * blk
    posf = rows.astype(jnp.float32)
    for f_ref, t_ref in ((f2a_ref, ta_ref), (f2b_ref, tb_ref), (f2c_ref, tc_ref)):
        w = t_ref.shape[1]
        ang = posf * f_ref[...]
        par = lax.broadcasted_iota(jnp.int32, (blk, w), 1)
        t_ref[...] = jnp.where(par % 2 == 0, jnp.cos(ang), jnp.sin(ang))


def _build_tables(f2a, f2b, f2c):
    blk = 1024
    return pl.pallas_call(
        _table_body,
        grid=(MAX_POS // blk,),
        in_specs=[pl.BlockSpec((1, w), lambda i: (0, 0)) for w in COLS],
        out_specs=[pl.BlockSpec((blk, w), lambda i: (i, 0)) for w in COLS],
        out_shape=[jax.ShapeDtypeStruct((MAX_POS, w), jnp.float32) for w in COLS],
    )(f2a, f2b, f2c)


_MESH = plsc.VectorSubcoreMesh(core_axis_name="c", subcore_axis_name="s")


@functools.partial(
    pl.kernel,
    mesh=_MESH,
    out_type=jax.ShapeDtypeStruct((BATCH, MAX_POS, HEAD_DIM), jnp.float32),
    scratch_types=[
        pltpu.VMEM((NG, GCH), jnp.int32),
        pltpu.VMEM((RPW, 32), jnp.float32),
        pltpu.VMEM((RPW, 48), jnp.float32),
        pltpu.SemaphoreType.DMA,
    ],
    compiler_params=pltpu.CompilerParams(use_tc_tiling_on_sc=False),
)
def _sc_gather(ta, tb, tc_, ids, out, idx_v, buf32, buf48, sem):
    wid = lax.axis_index("s") * NC + lax.axis_index("c")
    b = wid // WPB
    t0 = (wid % WPB) * RPW
    g0 = (wid % WPB) * NG
    for sec, tbl, col, buf in ((0, ta, 0, buf32), (1, tb, 32, buf48), (2, tc_, 80, buf48)):
        w = buf.shape[1]
        pltpu.sync_copy(ids.at[b, sec, pl.ds(g0, NG)], idx_v)
        cps = [
            pltpu.async_copy(tbl.at[idx_v.at[j]], buf.at[pl.ds(j * GCH, GCH)], sem)
            for j in range(NG)
        ]
        for cp in cps:
            cp.wait()
        pltpu.sync_copy(buf, out.at[b, pl.ds(t0, RPW), pl.ds(col, w)])


def kernel(mrope_position_ids_padding, mrope_position_deltas, inv_freq):
    f2a = jnp.repeat(inv_freq[0:16], 2)[None, :]
    f2b = jnp.repeat(inv_freq[16:40], 2)[None, :]
    f2c = jnp.repeat(inv_freq[40:64], 2)[None, :]
    ta, tb, tc_ = _build_tables(f2a, f2b, f2c)
    ids4 = mrope_position_ids_padding.reshape(BATCH, 3, MAX_POS // GCH, GCH)
    out = _sc_gather(ta, tb, tc_, ids4)
    return out.reshape(BATCH, MAX_POS * HEAD_DIM), mrope_position_deltas
